# Initial kernel scaffold; baseline (speedup 1.0000x reference)
import jax
import jax.numpy as jnp
from jax.experimental import pallas as pl


def kernel(features, params):
    raise NotImplementedError("write your pallas kernel here")


# speedup vs baseline: 9.4044x; 9.4044x over previous
import functools

import jax
import jax.numpy as jnp
from jax.experimental import pallas as pl

_N = 120
_TILE = 2000


def _dot(a, b):
    return jax.lax.dot_general(a, b, (((1,), (0,)), ((), ())),
                               preferred_element_type=jnp.float32)


def _dott(a, b):
    return jax.lax.dot_general(a, b, (((0,), (0,)), ((), ())),
                               preferred_element_type=jnp.float32)


def _relu(x):
    return jnp.maximum(x, 0.0)


def _flatten_params(params):
    out = []
    for blk in ('p1', 'p2', 'v1', 'v2'):
        mods = ('edge', 'node_mlp1', 'node_mlp2', 'global')
        if blk in ('p2', 'v2'):
            mods = ('edge', 'node_mlp1', 'node_mlp2')
        for m in mods:
            for (W, b) in params[blk][m]:
                out.append(W)
                out.append(b.reshape(1, -1))
    return out


def _body(nt, x0r, u0r, srcr, dstr, ear, *rest):
    prefs = list(rest[:-2])
    polr, valr = rest[-2:]

    vals = [r[...] for r in prefs]
    cursor = [0]

    def take(n):
        v = vals[cursor[0]:cursor[0] + n]
        cursor[0] += n
        return v

    blocks = {}
    for blk in ('p1', 'p2', 'v1', 'v2'):
        mods = ('edge', 'node_mlp1', 'node_mlp2', 'global')
        if blk in ('p2', 'v2'):
            mods = ('edge', 'node_mlp1', 'node_mlp2')
        d = {}
        for m in mods:
            d[m] = take(4)
        blocks[blk] = d

    x0 = x0r[...]
    u0 = u0r[...]

    def layer1_tables(blk):
        eW1, eb1, eW2, eb2 = blocks[blk]['edge']
        nW1, nb1, nW2, nb2 = blocks[blk]['node_mlp1']
        t = {}
        t['A'] = _dot(x0, eW1[0:5, :])
        t['B'] = _dot(x0, eW1[5:10, :])
        t['wc'] = eW1[10:11, :]
        t['U'] = _dot(u0, eW1[11:17, :]) + eb1
        t['C'] = _dot(x0, nW1[0:5, :])
        t['Nf'] = _dot(eW2, nW1[5:7, :])
        t['cf'] = nb1 + _dot(eb2, nW1[5:7, :])
        t['eW2'] = eW2
        t['eb2'] = eb2
        t['nW2'] = nW2
        t['nb2'] = nb2
        return t

    tp1 = layer1_tables('p1')
    tv1 = layer1_tables('v1')

    iota = jax.lax.broadcasted_iota(jnp.int32, (128, _TILE), 0)

    def onehots(t):
        srow = srcr[t]
        drow = dstr[t]
        ohs = (iota == srow).astype(jnp.float32)
        ohd = (iota == drow).astype(jnp.float32)
        return ohs, ohd

    def h1_of(t1, ohs, ohd, erow):
        return _relu(_dott(ohs, t1['A']) + _dott(ohd, t1['B'])
                     + _dott(erow, t1['wc']) + t1['U'])

    def pass1_body(t, carry):
        Sp, Sv, cnt = carry
        ohs, ohd = onehots(t)
        erow = ear[t]
        h1p = h1_of(tp1, ohs, ohd, erow)
        g1p = _relu(_dot(h1p, tp1['Nf']) + _dott(ohd, tp1['C']) + tp1['cf'])
        Sp = Sp + _dot(ohd, g1p)
        h1v = h1_of(tv1, ohs, ohd, erow)
        g1v = _relu(_dot(h1v, tv1['Nf']) + _dott(ohd, tv1['C']) + tv1['cf'])
        Sv = Sv + _dot(ohd, g1v)
        cnt = cnt + jnp.sum(ohd, axis=1, keepdims=True)
        return Sp, Sv, cnt

    zero128 = jnp.zeros((128, 128), jnp.float32)
    Sp, Sv, cnt = jax.lax.fori_loop(
        0, nt, pass1_body, (zero128, zero128, jnp.zeros((128, 1), jnp.float32)))

    cnt_safe = jnp.maximum(cnt, 1.0)
    mask = (jax.lax.broadcasted_iota(jnp.int32, (128, 1), 0) < _N).astype(jnp.float32)

    def node_stage(blk1, blk2, t1, S):
        n2W1, n2b1, n2W2, n2b2 = blocks[blk1]['node_mlp2']
        gW1, gb1, gW2, gb2 = blocks[blk1]['global']
        agg = _dot(S, t1['nW2']) + cnt * t1['nb2']
        aggm = agg / cnt_safe
        z = _relu(_dot(x0, n2W1[0:5, :]) + _dot(aggm, n2W1[5:133, :])
                  + cnt * n2W1[133:134, :] + n2b1)
        x1 = _dot(z, n2W2) + n2b2
        xm = jnp.sum(x1 * mask, axis=0, keepdims=True) * (1.0 / _N)
        u1 = _dot(_relu(_dot(u0, gW1[0:6, :]) + _dot(xm, gW1[6:16, :]) + gb1),
                  gW2) + gb2
        eW1, eb1, eW2, eb2 = blocks[blk2]['edge']
        mW1, mb1, mW2, mb2 = blocks[blk2]['node_mlp1']
        t2 = {}
        t2['A'] = _dot(x1, eW1[0:10, :])
        t2['B'] = _dot(x1, eW1[10:20, :])
        t2['M'] = _dot(t1['eW2'], eW1[20:22, :])
        t2['U'] = _dot(u1, eW1[22:34, :]) + eb1 + _dot(t1['eb2'], eW1[20:22, :])
        t2['C'] = _dot(x1, mW1[0:10, :])
        t2['Nf'] = _dot(eW2, mW1[10:11, :])
        t2['cf'] = mb1 + _dot(eb2, mW1[10:11, :])
        t2['mW2'] = mW2
        t2['mb2'] = mb2
        return x1, t2

    x1p, tp2 = node_stage('p1', 'p2', tp1, Sp)
    x1v, tv2 = node_stage('v1', 'v2', tv1, Sv)

    def pass2_body(t, carry):
        S2p, S2v = carry
        ohs, ohd = onehots(t)
        erow = ear[t]
        h1p = h1_of(tp1, ohs, ohd, erow)
        h2p = _relu(_dot(h1p, tp2['M']) + _dott(ohs, tp2['A'])
                    + _dott(ohd, tp2['B']) + tp2['U'])
        g2p = _relu(_dot(h2p, tp2['Nf']) + _dott(ohd, tp2['C']) + tp2['cf'])
        S2p = S2p + _dot(ohd, g2p)
        h1v = h1_of(tv1, ohs, ohd, erow)
        h2v = _relu(_dot(h1v, tv2['M']) + _dott(ohs, tv2['A'])
                    + _dott(ohd, tv2['B']) + tv2['U'])
        g2v = _relu(_dot(h2v, tv2['Nf']) + _dott(ohd, tv2['C']) + tv2['cf'])
        S2v = S2v + _dot(ohd, g2v)
        return S2p, S2v

    S2p, S2v = jax.lax.fori_loop(0, nt, pass2_body, (zero128, zero128))

    def final_stage(blk2, x1, t2, S2):
        q2W1, q2b1, q2W2, q2b2 = blocks[blk2]['node_mlp2']
        agg = _dot(S2, t2['mW2']) + cnt * t2['mb2']
        aggm = agg / cnt_safe
        z = _relu(_dot(x1, q2W1[0:10, :]) + _dot(aggm, q2W1[10:138, :])
                  + cnt * q2W1[138:139, :] + q2b1)
        return _dot(z, q2W2) + q2b2

    polr[...] = final_stage('p2', x1p, tp2, S2p)
    valr[...] = final_stage('v2', x1v, tv2, S2v)


def kernel(features, params):
    f = features[0]
    nodes = _N
    deg = f[0:nodes]
    cap = f[nodes:2 * nodes]
    inc = f[2 * nodes:3 * nodes]
    outg = f[3 * nodes:4 * nodes]
    tot = f[4 * nodes:5 * nodes]
    x0 = jnp.stack([cap, deg, inc, outg, tot], axis=1)
    x0 = jnp.pad(x0, ((0, 128 - nodes), (0, 0)))
    base = 5 * nodes + 6
    u0 = f[5 * nodes:base].reshape(1, 6)
    ne = (features.shape[1] - base) // 3
    nt = -(-ne // _TILE)
    pad = nt * _TILE - ne
    ea = f[base:base + ne]
    src = f[base + ne:base + 2 * ne].astype(jnp.int32)
    dst = f[base + 2 * ne:base + 3 * ne].astype(jnp.int32)
    if pad:
        ea = jnp.pad(ea, (0, pad))
        src = jnp.pad(src, (0, pad), constant_values=127)
        dst = jnp.pad(dst, (0, pad), constant_values=127)
    ea = ea.reshape(nt, 1, _TILE)
    src = src.reshape(nt, 1, _TILE)
    dst = dst.reshape(nt, 1, _TILE)

    plist = _flatten_params(params)
    pol, val = pl.pallas_call(
        functools.partial(_body, nt),
        out_shape=[jax.ShapeDtypeStruct((128, 1), jnp.float32),
                   jax.ShapeDtypeStruct((128, 1), jnp.float32)],
    )(x0, u0, src, dst, ea, *plist)
    policy = pol[:nodes, 0].reshape(1, nodes)
    value = val[:nodes, 0].reshape(1, nodes)
    return policy, value
